# baseline (speedup 1.0000x reference)
import functools
import math

import jax
import jax.numpy as jnp
from jax import lax
from jax.experimental import pallas as pl
from jax.experimental.pallas import tpu as pltpu
from jax.experimental.pallas import tpu_sc as plsc

_I32_MIN = -(2**31)
_NROWS = 512
_V = 8192
_K = math.ceil((1.0 - 0.9) * _V)
_NV = _V // 16
_NBINS = 128
_NGRP = _NBINS // 16


def _i32(x):
    return x.astype(jnp.int32) if hasattr(x, "astype") else jnp.int32(x)


def _digit(v):
    t = lax.convert_element_type(v * 16.0, jnp.int32)
    return jnp.clip(t + 64, 0, _NBINS - 1)


def _key_of(v):
    b = lax.bitcast_convert_type(v, jnp.int32)
    return jnp.where(b < 0, _I32_MIN - b, b)


def _sc_body(x_hbm, o_hbm, xv, ov, cbuf, hist, sem_in, sem_out):
    wid = lax.axis_index("s") * 2 + lax.axis_index("c")
    ii = lax.broadcasted_iota(jnp.int32, (16,), 0)
    ones16 = jnp.full((16,), 1, jnp.int32)

    def do_row(rr, _):
        row = wid * 16 + rr
        pltpu.async_copy(x_hbm.at[pl.ds(row * _V, _V)], xv, sem_in).wait()

        def zero_hist(i, _):
            hist[pl.ds(i * 16, 16)] = jnp.zeros((16,), jnp.int32)
            return 0

        lax.fori_loop(0, _NBINS * 16 // 16, zero_hist, 0)

        def pass_a(j, _):
            v = xv[pl.ds(j * 16, 16)]
            addr = ii * _NBINS + _digit(v)
            plsc.addupdate_scatter(hist, [addr], ones16)
            return 0

        lax.fori_loop(0, _NV, pass_a, 0)

        cum = jnp.int32(0)
        found = jnp.int32(0)
        b0 = jnp.int32(0)
        r1 = jnp.int32(1)
        for g in range(_NGRP - 1, -1, -1):
            def merge_l(l, acc):
                return acc + hist[pl.ds(l * _NBINS + 16 * g, 16)]

            mg = lax.fori_loop(1, 16, merge_l, hist[pl.ds(16 * g, 16)])
            rev = lax.rev(mg, (0,))
            cs = plsc.cumsum(rev)
            tot = jnp.max(cs)
            hit = cs >= (_K - cum)
            p = jnp.max(plsc.all_reduce_ffs(hit))
            in_this = jnp.logical_and(found == 0, cum + tot >= _K)
            cnt_d = jnp.sum(jnp.where(ii == p, rev, 0))
            cs_p = jnp.sum(jnp.where(ii == p, cs, 0))
            cum_above = cum + cs_p - cnt_d
            b0 = jnp.where(in_this, 16 * g + 15 - p, b0)
            r1 = jnp.where(in_this, _K - cum_above, r1)
            found = jnp.where(in_this, jnp.int32(1), found)
            cum = cum + tot

        def pass_c(j, ptr):
            v = xv[pl.ds(j * 16, 16)]
            m = _digit(v) == b0
            plsc.store_compressed(cbuf.at[pl.ds(ptr, 16)], _key_of(v), mask=m)
            return ptr + jnp.max(plsc.all_reduce_population_count(m))

        n1 = lax.fori_loop(0, _NV, pass_c, jnp.int32(0))
        cbuf[pl.ds(n1, 16)] = jnp.full((16,), _I32_MIN, jnp.int32)
        nv1 = n1 // 16 + 1

        def bs(i, t_u):
            cand_u = t_u | lax.shift_left(jnp.int32(1), 31 - i)
            cand_s = cand_u ^ _I32_MIN

            def cnt_body(j, acc):
                kv = cbuf[pl.ds(j * 16, 16)]
                return acc + (kv >= cand_s).astype(jnp.int32)

            acc = lax.fori_loop(0, nv1, cnt_body, jnp.zeros((16,), jnp.int32))
            cnt = jnp.sum(acc)
            return jnp.where(cnt >= r1, cand_u, t_u)

        t_u = lax.fori_loop(0, 32, bs, jnp.int32(0))
        t_s = t_u ^ _I32_MIN

        def stats_body(j, accs):
            a_gt, a_eq = accs
            kv = cbuf[pl.ds(j * 16, 16)]
            return (a_gt + (kv > t_s).astype(jnp.int32),
                    a_eq + (kv == t_s).astype(jnp.int32))

        z16 = jnp.zeros((16,), jnp.int32)
        a_gt, a_eq = lax.fori_loop(0, nv1, stats_body, (z16, z16))
        c_gt = jnp.sum(a_gt)
        n_eq = jnp.sum(a_eq)
        e = r1 - c_gt

        @pl.when(e == n_eq)
        def _fast():
            def out_b(j, _):
                v = xv[pl.ds(j * 16, 16)]
                keep = _key_of(v) >= t_s
                ov[pl.ds(j * 16, 16)] = jnp.where(keep, v, -jnp.inf)
                return 0

            lax.fori_loop(0, _NV, out_b, 0)

        @pl.when(e != n_eq)
        def _slow():
            def out_b(j, run):
                v = xv[pl.ds(j * 16, 16)]
                key = _key_of(v)
                eqm = key == t_s
                cs = plsc.cumsum(eqm.astype(jnp.int32))
                keep = (key > t_s) | (eqm & ((run + cs) <= e))
                ov[pl.ds(j * 16, 16)] = jnp.where(keep, v, -jnp.inf)
                return run + jnp.max(cs)

            lax.fori_loop(0, _NV, out_b, jnp.int32(0))

        pltpu.async_copy(ov, o_hbm.at[pl.ds(row * _V, _V)], sem_out).wait()
        return 0

    lax.fori_loop(0, _NROWS // 32, do_row, 0)


@jax.jit
def _sc_topk_mask(flat):
    mesh = plsc.VectorSubcoreMesh(core_axis_name="c", subcore_axis_name="s")
    return pl.kernel(
        _sc_body,
        out_type=jax.ShapeDtypeStruct((_NROWS * _V,), jnp.float32),
        mesh=mesh,
        compiler_params=pltpu.CompilerParams(needs_layout_passes=False),
        scratch_types=[
            pltpu.VMEM((_V,), jnp.float32),
            pltpu.VMEM((_V,), jnp.float32),
            pltpu.VMEM((_V + 16,), jnp.int32),
            pltpu.VMEM((_NBINS * 16,), jnp.int32),
            pltpu.SemaphoreType.DMA,
            pltpu.SemaphoreType.DMA,
        ],
    )(flat)


def kernel(logits):
    B, S, V = logits.shape
    out = _sc_topk_mask(logits.reshape(-1))
    return out.reshape(B, S, V)


# speedup vs baseline: 1.1624x; 1.1624x over previous
import functools
import math

import jax
import jax.numpy as jnp
from jax import lax
from jax.experimental import pallas as pl
from jax.experimental.pallas import tpu as pltpu
from jax.experimental.pallas import tpu_sc as plsc

_I32_MIN = -(2**31)
_NROWS = 512
_V = 8192
_K = math.ceil((1.0 - 0.9) * _V)
_NV = _V // 16
_NBINS = 128
_NGRP = _NBINS // 16
_UNROLL = 8


def _digit(v):
    t = lax.convert_element_type(v * 16.0, jnp.int32)
    return jnp.clip(t + 64, 0, _NBINS - 1)


def _key_of(v):
    b = lax.bitcast_convert_type(v, jnp.int32)
    return jnp.where(b < 0, _I32_MIN - b, b)


def _sc_body(x_hbm, o_hbm, xv0, xv1, ov0, ov1, cbuf, hist,
             si0, si1, so0, so1):
    wid = lax.axis_index("s") * 2 + lax.axis_index("c")
    ii = lax.broadcasted_iota(jnp.int32, (16,), 0)
    bank = ii * _NBINS
    ones16 = jnp.full((16,), 1, jnp.int32)

    def row_slice(rr):
        return x_hbm.at[pl.ds((wid * 16 + rr) * _V, _V)]

    def out_slice(rr):
        return o_hbm.at[pl.ds((wid * 16 + rr) * _V, _V)]

    def select_row(xv, ov):
        def zero_hist(i, _):
            for c in range(_UNROLL):
                hist[pl.ds((i * _UNROLL + c) * 16, 16)] = jnp.zeros(
                    (16,), jnp.int32)
            return 0

        lax.fori_loop(0, _NBINS * 16 // 16 // _UNROLL, zero_hist, 0)

        def pass_a(j, _):
            for c in range(_UNROLL):
                v = xv[pl.ds((j * _UNROLL + c) * 16, 16)]
                plsc.addupdate_scatter(hist, [bank + _digit(v)], ones16)
            return 0

        lax.fori_loop(0, _NV // _UNROLL, pass_a, 0)

        cum = jnp.int32(0)
        found = jnp.int32(0)
        b0 = jnp.int32(0)
        r1 = jnp.int32(1)
        for g in range(_NGRP - 1, -1, -1):
            mg = hist[pl.ds(16 * g, 16)]
            for l in range(1, 16):
                mg = mg + hist[pl.ds(l * _NBINS + 16 * g, 16)]
            rev = lax.rev(mg, (0,))
            cs = plsc.cumsum(rev)
            tot = jnp.max(cs)
            hit = cs >= (_K - cum)
            p = jnp.max(plsc.all_reduce_ffs(hit))
            in_this = jnp.logical_and(found == 0, cum + tot >= _K)
            cnt_d = jnp.sum(jnp.where(ii == p, rev, 0))
            cs_p = jnp.sum(jnp.where(ii == p, cs, 0))
            cum_above = cum + cs_p - cnt_d
            b0 = jnp.where(in_this, 16 * g + 15 - p, b0)
            r1 = jnp.where(in_this, _K - cum_above, r1)
            found = jnp.where(in_this, jnp.int32(1), found)
            cum = cum + tot

        def pass_c(j, ptr):
            for c in range(_UNROLL):
                v = xv[pl.ds((j * _UNROLL + c) * 16, 16)]
                m = _digit(v) == b0
                plsc.store_compressed(cbuf.at[pl.ds(ptr, 16)], _key_of(v),
                                      mask=m)
                ptr = ptr + jnp.max(plsc.all_reduce_population_count(m))
            return ptr

        n1 = lax.fori_loop(0, _NV // _UNROLL, pass_c, jnp.int32(0))
        pad = jnp.full((16,), _I32_MIN, jnp.int32)
        for c in range(4):
            cbuf[pl.ds(n1 + 16 * c, 16)] = pad
        nv4 = lax.shift_right_logical(n1 + 63, 6)

        def bs(i, t_u):
            cand_u = t_u | lax.shift_left(jnp.int32(1), 31 - i)
            cand_s = cand_u ^ _I32_MIN

            def cnt_body(j, acc):
                for c in range(4):
                    kv = cbuf[pl.ds(j * 64 + c * 16, 16)]
                    acc = acc + (kv >= cand_s).astype(jnp.int32)
                return acc

            acc = lax.fori_loop(0, nv4, cnt_body, jnp.zeros((16,), jnp.int32))
            cnt = jnp.sum(acc)
            return jnp.where(cnt >= r1, cand_u, t_u)

        t_u = lax.fori_loop(0, 32, bs, jnp.int32(0))
        t_s = t_u ^ _I32_MIN

        def stats_body(j, accs):
            a_gt, a_eq = accs
            for c in range(4):
                kv = cbuf[pl.ds(j * 64 + c * 16, 16)]
                a_gt = a_gt + (kv > t_s).astype(jnp.int32)
                a_eq = a_eq + (kv == t_s).astype(jnp.int32)
            return a_gt, a_eq

        z16 = jnp.zeros((16,), jnp.int32)
        a_gt, a_eq = lax.fori_loop(0, nv4, stats_body, (z16, z16))
        c_gt = jnp.sum(a_gt)
        n_eq = jnp.sum(a_eq)
        e = r1 - c_gt

        @pl.when(e == n_eq)
        def _fast():
            def out_b(j, _):
                for c in range(_UNROLL):
                    sl = pl.ds((j * _UNROLL + c) * 16, 16)
                    v = xv[sl]
                    ov[sl] = jnp.where(_key_of(v) >= t_s, v, -jnp.inf)
                return 0

            lax.fori_loop(0, _NV // _UNROLL, out_b, 0)

        @pl.when(e != n_eq)
        def _slow():
            def out_b(j, run):
                for c in range(_UNROLL):
                    sl = pl.ds((j * _UNROLL + c) * 16, 16)
                    v = xv[sl]
                    key = _key_of(v)
                    eqm = key == t_s
                    cs = plsc.cumsum(eqm.astype(jnp.int32))
                    keep = (key > t_s) | (eqm & ((run + cs) <= e))
                    ov[sl] = jnp.where(keep, v, -jnp.inf)
                    run = run + plsc.all_reduce_population_count(eqm)
                return run

            lax.fori_loop(0, _NV // _UNROLL, out_b, jnp.zeros((16,), jnp.int32))

    pltpu.async_copy(row_slice(0), xv0, si0)

    def chunk(i, _):
        r0 = 2 * i
        pltpu.async_copy(row_slice(r0 + 1), xv1, si1)
        pltpu.make_async_copy(row_slice(r0), xv0, si0).wait()

        @pl.when(i > 0)
        def _w0():
            pltpu.make_async_copy(ov0, out_slice(2 * i - 2), so0).wait()

        select_row(xv0, ov0)
        pltpu.async_copy(ov0, out_slice(r0), so0)

        @pl.when(i < 7)
        def _n1():
            pltpu.async_copy(row_slice(r0 + 2), xv0, si0)

        pltpu.make_async_copy(row_slice(r0 + 1), xv1, si1).wait()

        @pl.when(i > 0)
        def _w1():
            pltpu.make_async_copy(ov1, out_slice(2 * i - 1), so1).wait()

        select_row(xv1, ov1)
        pltpu.async_copy(ov1, out_slice(r0 + 1), so1)
        return 0

    lax.fori_loop(0, 8, chunk, 0)
    pltpu.make_async_copy(ov0, out_slice(14), so0).wait()
    pltpu.make_async_copy(ov1, out_slice(15), so1).wait()


@jax.jit
def _sc_topk_mask(flat):
    mesh = plsc.VectorSubcoreMesh(core_axis_name="c", subcore_axis_name="s")
    return pl.kernel(
        _sc_body,
        out_type=jax.ShapeDtypeStruct((_NROWS * _V,), jnp.float32),
        mesh=mesh,
        compiler_params=pltpu.CompilerParams(needs_layout_passes=False),
        scratch_types=[
            pltpu.VMEM((_V,), jnp.float32),
            pltpu.VMEM((_V,), jnp.float32),
            pltpu.VMEM((_V,), jnp.float32),
            pltpu.VMEM((_V,), jnp.float32),
            pltpu.VMEM((_V + 64,), jnp.int32),
            pltpu.VMEM((_NBINS * 16,), jnp.int32),
            pltpu.SemaphoreType.DMA,
            pltpu.SemaphoreType.DMA,
            pltpu.SemaphoreType.DMA,
            pltpu.SemaphoreType.DMA,
        ],
    )(flat)


def kernel(logits):
    B, S, V = logits.shape
    out = _sc_topk_mask(logits.reshape(-1))
    return out.reshape(B, S, V)


# speedup vs baseline: 1.5518x; 1.3350x over previous
import functools
import math

import jax
import jax.numpy as jnp
from jax import lax
from jax.experimental import pallas as pl
from jax.experimental.pallas import tpu as pltpu
from jax.experimental.pallas import tpu_sc as plsc

_I32_MIN = -(2**31)
_NROWS = 512
_V = 8192
_K = math.ceil((1.0 - 0.9) * _V)
_NV = _V // 16
_NBINS = 64
_NGRP = _NBINS // 16
_UNROLL = 8
_FAST_CAP = 240


def _digit(v):
    t = lax.convert_element_type(v * 8.0, jnp.int32)
    return jnp.clip(t + 32, 0, _NBINS - 1)


def _key_of(v):
    b = lax.bitcast_convert_type(v, jnp.int32)
    return jnp.where(b < 0, _I32_MIN - b, b)


def _sc_body(x_hbm, o_hbm, xv0, xv1, ov0, ov1, cbuf, hist,
             si0, si1, so0, so1):
    wid = lax.axis_index("s") * 2 + lax.axis_index("c")
    ii = lax.broadcasted_iota(jnp.int32, (16,), 0)
    ones16 = jnp.full((16,), 1, jnp.int32)
    pad16 = jnp.full((16,), _I32_MIN, jnp.int32)

    def row_slice(rr):
        return x_hbm.at[pl.ds((wid * 16 + rr) * _V, _V)]

    def out_slice(rr):
        return o_hbm.at[pl.ds((wid * 16 + rr) * _V, _V)]

    def select_row(xv, ov):
        for c in range(16):
            cbuf[pl.ds(c * 16, 16)] = pad16

        def zero_hist(i, _):
            for c in range(_UNROLL):
                hist[pl.ds((i * _UNROLL + c) * 16, 16)] = jnp.zeros(
                    (16,), jnp.int32)
            return 0

        lax.fori_loop(0, _NBINS * 16 // 16 // _UNROLL, zero_hist, 0)

        def pass_a(j, _):
            for c in range(_UNROLL):
                v = xv[pl.ds((j * _UNROLL + c) * 16, 16)]
                addr = lax.shift_left(_digit(v), 4) + ii
                plsc.addupdate_scatter(hist, [addr], ones16)
            return 0

        lax.fori_loop(0, _NV // _UNROLL, pass_a, 0)

        cum = jnp.int32(0)
        found = jnp.int32(0)
        b0 = jnp.int32(0)
        r1 = jnp.int32(1)
        for g in range(_NGRP - 1, -1, -1):
            mg = jnp.zeros((16,), jnp.int32)
            for b in range(16):
                s_b = jnp.sum(hist[pl.ds((16 * g + b) * 16, 16)])
                mg = jnp.where(ii == b, s_b, mg)
            rev = lax.rev(mg, (0,))
            cs = plsc.cumsum(rev)
            tot = jnp.max(cs)
            hit = cs >= (_K - cum)
            p = jnp.max(plsc.all_reduce_ffs(hit))
            in_this = jnp.logical_and(found == 0, cum + tot >= _K)
            cnt_d = jnp.sum(jnp.where(ii == p, rev, 0))
            cs_p = jnp.sum(jnp.where(ii == p, cs, 0))
            cum_above = cum + cs_p - cnt_d
            b0 = jnp.where(in_this, 16 * g + 15 - p, b0)
            r1 = jnp.where(in_this, _K - cum_above, r1)
            found = jnp.where(in_this, jnp.int32(1), found)
            cum = cum + tot

        def pass_c(j, ptr):
            ms, keys, cnts = [], [], []
            for c in range(_UNROLL):
                v = xv[pl.ds((j * _UNROLL + c) * 16, 16)]
                m = _digit(v) == b0
                ms.append(m)
                keys.append(_key_of(v))
                cnts.append(jnp.max(plsc.all_reduce_population_count(m)))
            off = ptr
            for c in range(_UNROLL):
                plsc.store_compressed(cbuf.at[pl.ds(off, 16)], keys[c],
                                      mask=ms[c])
                off = off + cnts[c]
            return off

        n1 = lax.fori_loop(0, _NV // _UNROLL, pass_c, jnp.int32(0))

        def search_fast():
            kvs = [cbuf[pl.ds(c * 16, 16)] for c in range(16)]
            t_u = jnp.int32(0)
            for bit in range(31, -1, -1):
                bconst = -(1 << 31) if bit == 31 else (1 << bit)
                cand_u = t_u | jnp.int32(bconst)
                cand_s = cand_u ^ _I32_MIN
                acc = jnp.zeros((16,), jnp.int32)
                for c in range(16):
                    acc = acc + (kvs[c] >= cand_s).astype(jnp.int32)
                t_u = jnp.where(jnp.sum(acc) >= r1, cand_u, t_u)
            t_s = t_u ^ _I32_MIN
            a_gt = jnp.zeros((16,), jnp.int32)
            a_eq = jnp.zeros((16,), jnp.int32)
            for c in range(16):
                a_gt = a_gt + (kvs[c] > t_s).astype(jnp.int32)
                a_eq = a_eq + (kvs[c] == t_s).astype(jnp.int32)
            return t_s, jnp.sum(a_gt), jnp.sum(a_eq)

        def search_slow():
            for c in range(4):
                cbuf[pl.ds(n1 + 16 * c, 16)] = pad16
            nv4 = lax.shift_right_logical(n1 + 63, 6)

            def bs(i, t_u):
                cand_u = t_u | lax.shift_left(jnp.int32(1), 31 - i)
                cand_s = cand_u ^ _I32_MIN

                def cnt_body(j, acc):
                    for c in range(4):
                        kv = cbuf[pl.ds(j * 64 + c * 16, 16)]
                        acc = acc + (kv >= cand_s).astype(jnp.int32)
                    return acc

                acc = lax.fori_loop(0, nv4, cnt_body,
                                    jnp.zeros((16,), jnp.int32))
                return jnp.where(jnp.sum(acc) >= r1, cand_u, t_u)

            t_u = lax.fori_loop(0, 32, bs, jnp.int32(0))
            t_s = t_u ^ _I32_MIN

            def stats_body(j, accs):
                a_gt, a_eq = accs
                for c in range(4):
                    kv = cbuf[pl.ds(j * 64 + c * 16, 16)]
                    a_gt = a_gt + (kv > t_s).astype(jnp.int32)
                    a_eq = a_eq + (kv == t_s).astype(jnp.int32)
                return a_gt, a_eq

            z16 = jnp.zeros((16,), jnp.int32)
            a_gt, a_eq = lax.fori_loop(0, nv4, stats_body, (z16, z16))
            return t_s, jnp.sum(a_gt), jnp.sum(a_eq)

        t_s, c_gt, n_eq = lax.cond(n1 <= _FAST_CAP, search_fast, search_slow)
        e = r1 - c_gt

        @pl.when(e == n_eq)
        def _fast():
            def out_b(j, _):
                for c in range(_UNROLL):
                    sl = pl.ds((j * _UNROLL + c) * 16, 16)
                    v = xv[sl]
                    ov[sl] = jnp.where(_key_of(v) >= t_s, v, -jnp.inf)
                return 0

            lax.fori_loop(0, _NV // _UNROLL, out_b, 0)

        @pl.when(e != n_eq)
        def _slow():
            def out_b(j, run):
                for c in range(_UNROLL):
                    sl = pl.ds((j * _UNROLL + c) * 16, 16)
                    v = xv[sl]
                    key = _key_of(v)
                    eqm = key == t_s
                    cs = plsc.cumsum(eqm.astype(jnp.int32))
                    keep = (key > t_s) | (eqm & ((run + cs) <= e))
                    ov[sl] = jnp.where(keep, v, -jnp.inf)
                    run = run + plsc.all_reduce_population_count(eqm)
                return run

            lax.fori_loop(0, _NV // _UNROLL, out_b,
                          jnp.zeros((16,), jnp.int32))

    pltpu.async_copy(row_slice(0), xv0, si0)

    def chunk(i, _):
        r0 = 2 * i
        pltpu.async_copy(row_slice(r0 + 1), xv1, si1)
        pltpu.make_async_copy(row_slice(r0), xv0, si0).wait()

        @pl.when(i > 0)
        def _w0():
            pltpu.make_async_copy(ov0, out_slice(2 * i - 2), so0).wait()

        select_row(xv0, ov0)
        pltpu.async_copy(ov0, out_slice(r0), so0)

        @pl.when(i < 7)
        def _n1():
            pltpu.async_copy(row_slice(r0 + 2), xv0, si0)

        pltpu.make_async_copy(row_slice(r0 + 1), xv1, si1).wait()

        @pl.when(i > 0)
        def _w1():
            pltpu.make_async_copy(ov1, out_slice(2 * i - 1), so1).wait()

        select_row(xv1, ov1)
        pltpu.async_copy(ov1, out_slice(r0 + 1), so1)
        return 0

    lax.fori_loop(0, 8, chunk, 0)
    pltpu.make_async_copy(ov0, out_slice(14), so0).wait()
    pltpu.make_async_copy(ov1, out_slice(15), so1).wait()


@jax.jit
def _sc_topk_mask(flat):
    mesh = plsc.VectorSubcoreMesh(core_axis_name="c", subcore_axis_name="s")
    return pl.kernel(
        _sc_body,
        out_type=jax.ShapeDtypeStruct((_NROWS * _V,), jnp.float32),
        mesh=mesh,
        compiler_params=pltpu.CompilerParams(needs_layout_passes=False),
        scratch_types=[
            pltpu.VMEM((_V,), jnp.float32),
            pltpu.VMEM((_V,), jnp.float32),
            pltpu.VMEM((_V,), jnp.float32),
            pltpu.VMEM((_V,), jnp.float32),
            pltpu.VMEM((_V + 64,), jnp.int32),
            pltpu.VMEM((_NBINS * 16,), jnp.int32),
            pltpu.SemaphoreType.DMA,
            pltpu.SemaphoreType.DMA,
            pltpu.SemaphoreType.DMA,
            pltpu.SemaphoreType.DMA,
        ],
    )(flat)


def kernel(logits):
    B, S, V = logits.shape
    out = _sc_topk_mask(logits.reshape(-1))
    return out.reshape(B, S, V)


# speedup vs baseline: 2.8364x; 1.8278x over previous
import functools
import math

import jax
import jax.numpy as jnp
from jax import lax
from jax.experimental import pallas as pl
from jax.experimental.pallas import tpu as pltpu
from jax.experimental.pallas import tpu_sc as plsc

_I32_MIN = -(2**31)
_NROWS = 512
_V = 8192
_K = math.ceil((1.0 - 0.9) * _V)
_NV = _V // 16
_NBINS = 64
_NGRP = _NBINS // 16
_UNROLL = 8
_FAST_CAP = 240


def _digit(v):
    t = lax.convert_element_type(v * 8.0, jnp.int32)
    return jnp.clip(t + 32, 0, _NBINS - 1)


def _key_of(v):
    b = lax.bitcast_convert_type(v, jnp.int32)
    return jnp.where(b < 0, _I32_MIN - b, b)


def _sc_body(x_hbm, o_hbm, xv0, xv1, ov0, ov1, cbuf, hist,
             si0, si1, so0, so1):
    wid = lax.axis_index("s") * 2 + lax.axis_index("c")
    ii = lax.broadcasted_iota(jnp.int32, (16,), 0)
    ones16 = jnp.full((16,), 1, jnp.int32)
    pad16 = jnp.full((16,), _I32_MIN, jnp.int32)

    def row_slice(rr):
        return x_hbm.at[pl.ds((wid * 16 + rr) * _V, _V)]

    def out_slice(rr):
        return o_hbm.at[pl.ds((wid * 16 + rr) * _V, _V)]

    def select_row(xv, ov):
        for c in range(16):
            cbuf[pl.ds(c * 16, 16)] = pad16

        @plsc.parallel_loop(0, _NBINS * 16 // 16, unroll=_UNROLL)
        def zero_hist(i):
            hist[pl.ds(i * 16, 16)] = jnp.zeros((16,), jnp.int32)

        @plsc.parallel_loop(0, _NV, unroll=_UNROLL)
        def pass_a(j):
            v = xv[pl.ds(j * 16, 16)]
            addr = lax.shift_left(_digit(v), 4) + ii
            plsc.addupdate_scatter(hist, [addr], ones16)

        cum = jnp.int32(0)
        found = jnp.int32(0)
        b0 = jnp.int32(0)
        r1 = jnp.int32(1)
        for g in range(_NGRP - 1, -1, -1):
            mg = jnp.zeros((16,), jnp.int32)
            for b in range(16):
                s_b = jnp.sum(hist[pl.ds((16 * g + b) * 16, 16)])
                mg = jnp.where(ii == b, s_b, mg)
            rev = lax.rev(mg, (0,))
            cs = plsc.cumsum(rev)
            tot = jnp.max(cs)
            hit = cs >= (_K - cum)
            p = jnp.max(plsc.all_reduce_ffs(hit))
            in_this = jnp.logical_and(found == 0, cum + tot >= _K)
            cnt_d = jnp.sum(jnp.where(ii == p, rev, 0))
            cs_p = jnp.sum(jnp.where(ii == p, cs, 0))
            cum_above = cum + cs_p - cnt_d
            b0 = jnp.where(in_this, 16 * g + 15 - p, b0)
            r1 = jnp.where(in_this, _K - cum_above, r1)
            found = jnp.where(in_this, jnp.int32(1), found)
            cum = cum + tot

        def pass_c(j, ptr):
            v = xv[pl.ds(j * 16, 16)]
            m = _digit(v) == b0
            plsc.store_compressed(cbuf.at[pl.ds(ptr, 16)], _key_of(v),
                                  mask=m)
            return ptr + jnp.max(plsc.all_reduce_population_count(m))

        n1 = plsc.parallel_loop(0, _NV, unroll=_UNROLL,
                                carry=jnp.int32(0))(pass_c)

        def search_fast():
            kvs = [cbuf[pl.ds(c * 16, 16)] for c in range(16)]
            t_u = jnp.int32(0)
            for bit in range(31, -1, -1):
                bconst = -(1 << 31) if bit == 31 else (1 << bit)
                cand_u = t_u | jnp.int32(bconst)
                cand_s = cand_u ^ _I32_MIN
                acc = jnp.zeros((16,), jnp.int32)
                for c in range(16):
                    acc = acc + (kvs[c] >= cand_s).astype(jnp.int32)
                t_u = jnp.where(jnp.sum(acc) >= r1, cand_u, t_u)
            t_s = t_u ^ _I32_MIN
            a_gt = jnp.zeros((16,), jnp.int32)
            a_eq = jnp.zeros((16,), jnp.int32)
            for c in range(16):
                a_gt = a_gt + (kvs[c] > t_s).astype(jnp.int32)
                a_eq = a_eq + (kvs[c] == t_s).astype(jnp.int32)
            return t_s, jnp.sum(a_gt), jnp.sum(a_eq)

        def search_slow():
            for c in range(4):
                cbuf[pl.ds(n1 + 16 * c, 16)] = pad16
            nv4 = lax.shift_right_logical(n1 + 63, 6)

            def bs(i, t_u):
                cand_u = t_u | lax.shift_left(jnp.int32(1), 31 - i)
                cand_s = cand_u ^ _I32_MIN

                def cnt_body(j, acc):
                    for c in range(4):
                        kv = cbuf[pl.ds(j * 64 + c * 16, 16)]
                        acc = acc + (kv >= cand_s).astype(jnp.int32)
                    return acc

                acc = lax.fori_loop(0, nv4, cnt_body,
                                    jnp.zeros((16,), jnp.int32))
                return jnp.where(jnp.sum(acc) >= r1, cand_u, t_u)

            t_u = lax.fori_loop(0, 32, bs, jnp.int32(0))
            t_s = t_u ^ _I32_MIN

            def stats_body(j, accs):
                a_gt, a_eq = accs
                for c in range(4):
                    kv = cbuf[pl.ds(j * 64 + c * 16, 16)]
                    a_gt = a_gt + (kv > t_s).astype(jnp.int32)
                    a_eq = a_eq + (kv == t_s).astype(jnp.int32)
                return a_gt, a_eq

            z16 = jnp.zeros((16,), jnp.int32)
            a_gt, a_eq = lax.fori_loop(0, nv4, stats_body, (z16, z16))
            return t_s, jnp.sum(a_gt), jnp.sum(a_eq)

        t_s, c_gt, n_eq = lax.cond(n1 <= _FAST_CAP, search_fast, search_slow)
        e = r1 - c_gt

        @pl.when(e == n_eq)
        def _fast():
            @plsc.parallel_loop(0, _NV, unroll=_UNROLL)
            def out_b(j):
                sl = pl.ds(j * 16, 16)
                v = xv[sl]
                ov[sl] = jnp.where(_key_of(v) >= t_s, v, -jnp.inf)

        @pl.when(e != n_eq)
        def _slow():
            def out_b(j, run):
                sl = pl.ds(j * 16, 16)
                v = xv[sl]
                key = _key_of(v)
                eqm = key == t_s
                cs = plsc.cumsum(eqm.astype(jnp.int32))
                keep = (key > t_s) | (eqm & ((run + cs) <= e))
                ov[sl] = jnp.where(keep, v, -jnp.inf)
                return run + plsc.all_reduce_population_count(eqm)

            plsc.parallel_loop(0, _NV, unroll=_UNROLL,
                               carry=jnp.zeros((16,), jnp.int32))(out_b)

    pltpu.async_copy(row_slice(0), xv0, si0)

    def chunk(i, _):
        r0 = 2 * i
        pltpu.async_copy(row_slice(r0 + 1), xv1, si1)
        pltpu.make_async_copy(row_slice(r0), xv0, si0).wait()

        @pl.when(i > 0)
        def _w0():
            pltpu.make_async_copy(ov0, out_slice(2 * i - 2), so0).wait()

        select_row(xv0, ov0)
        pltpu.async_copy(ov0, out_slice(r0), so0)

        @pl.when(i < 7)
        def _n1():
            pltpu.async_copy(row_slice(r0 + 2), xv0, si0)

        pltpu.make_async_copy(row_slice(r0 + 1), xv1, si1).wait()

        @pl.when(i > 0)
        def _w1():
            pltpu.make_async_copy(ov1, out_slice(2 * i - 1), so1).wait()

        select_row(xv1, ov1)
        pltpu.async_copy(ov1, out_slice(r0 + 1), so1)
        return 0

    lax.fori_loop(0, 8, chunk, 0)
    pltpu.make_async_copy(ov0, out_slice(14), so0).wait()
    pltpu.make_async_copy(ov1, out_slice(15), so1).wait()


@jax.jit
def _sc_topk_mask(flat):
    mesh = plsc.VectorSubcoreMesh(core_axis_name="c", subcore_axis_name="s")
    return pl.kernel(
        _sc_body,
        out_type=jax.ShapeDtypeStruct((_NROWS * _V,), jnp.float32),
        mesh=mesh,
        compiler_params=pltpu.CompilerParams(needs_layout_passes=False),
        scratch_types=[
            pltpu.VMEM((_V,), jnp.float32),
            pltpu.VMEM((_V,), jnp.float32),
            pltpu.VMEM((_V,), jnp.float32),
            pltpu.VMEM((_V,), jnp.float32),
            pltpu.VMEM((_V + 64,), jnp.int32),
            pltpu.VMEM((_NBINS * 16,), jnp.int32),
            pltpu.SemaphoreType.DMA,
            pltpu.SemaphoreType.DMA,
            pltpu.SemaphoreType.DMA,
            pltpu.SemaphoreType.DMA,
        ],
    )(flat)


def kernel(logits):
    B, S, V = logits.shape
    out = _sc_topk_mask(logits.reshape(-1))
    return out.reshape(B, S, V)


# speedup vs baseline: 2.8517x; 1.0054x over previous
import functools
import math

import jax
import jax.numpy as jnp
from jax import lax
from jax.experimental import pallas as pl
from jax.experimental.pallas import tpu as pltpu
from jax.experimental.pallas import tpu_sc as plsc

_I32_MIN = -(2**31)
_NROWS_SC = 256
_RPW = _NROWS_SC // 32
_V = 8192
_K = math.ceil((1.0 - 0.9) * _V)
_NV = _V // 16
_NBINS = 64
_NGRP = _NBINS // 16
_UNROLL = 8
_FAST_CAP = 240


def _digit(v):
    t = lax.convert_element_type(v * 8.0, jnp.int32)
    return jnp.clip(t + 32, 0, _NBINS - 1)


def _key_of(v):
    b = lax.bitcast_convert_type(v, jnp.int32)
    return jnp.where(b < 0, _I32_MIN - b, b)


def _sc_body(x_hbm, o_hbm, xv0, xv1, ov0, ov1, cbuf, hist,
             si0, si1, so0, so1):
    wid = lax.axis_index("s") * 2 + lax.axis_index("c")
    ii = lax.broadcasted_iota(jnp.int32, (16,), 0)
    ones16 = jnp.full((16,), 1, jnp.int32)
    pad16 = jnp.full((16,), _I32_MIN, jnp.int32)

    def row_slice(rr):
        return x_hbm.at[pl.ds((wid * _RPW + rr) * _V, _V)]

    def out_slice(rr):
        return o_hbm.at[pl.ds((wid * _RPW + rr) * _V, _V)]

    def select_row(xv, ov):
        for c in range(16):
            cbuf[pl.ds(c * 16, 16)] = pad16

        @plsc.parallel_loop(0, _NBINS * 16 // 16, unroll=_UNROLL)
        def zero_hist(i):
            hist[pl.ds(i * 16, 16)] = jnp.zeros((16,), jnp.int32)

        @plsc.parallel_loop(0, _NV, unroll=_UNROLL)
        def pass_a(j):
            v = xv[pl.ds(j * 16, 16)]
            addr = lax.shift_left(_digit(v), 4) + ii
            plsc.addupdate_scatter(hist, [addr], ones16)

        cum = jnp.int32(0)
        found = jnp.int32(0)
        b0 = jnp.int32(0)
        r1 = jnp.int32(1)
        for g in range(_NGRP - 1, -1, -1):
            mg = jnp.zeros((16,), jnp.int32)
            for b in range(16):
                s_b = jnp.sum(hist[pl.ds((16 * g + b) * 16, 16)])
                mg = jnp.where(ii == b, s_b, mg)
            rev = lax.rev(mg, (0,))
            cs = plsc.cumsum(rev)
            tot = jnp.max(cs)
            hit = cs >= (_K - cum)
            p = jnp.max(plsc.all_reduce_ffs(hit))
            in_this = jnp.logical_and(found == 0, cum + tot >= _K)
            cnt_d = jnp.sum(jnp.where(ii == p, rev, 0))
            cs_p = jnp.sum(jnp.where(ii == p, cs, 0))
            cum_above = cum + cs_p - cnt_d
            b0 = jnp.where(in_this, 16 * g + 15 - p, b0)
            r1 = jnp.where(in_this, _K - cum_above, r1)
            found = jnp.where(in_this, jnp.int32(1), found)
            cum = cum + tot

        def pass_c(j, ptr):
            v = xv[pl.ds(j * 16, 16)]
            m = _digit(v) == b0
            plsc.store_compressed(cbuf.at[pl.ds(ptr, 16)], _key_of(v),
                                  mask=m)
            return ptr + jnp.max(plsc.all_reduce_population_count(m))

        n1 = plsc.parallel_loop(0, _NV, unroll=_UNROLL,
                                carry=jnp.int32(0))(pass_c)

        def search_fast():
            kvs = [cbuf[pl.ds(c * 16, 16)] for c in range(16)]
            t_u = jnp.int32(0)
            for bit in range(31, -1, -1):
                bconst = -(1 << 31) if bit == 31 else (1 << bit)
                cand_u = t_u | jnp.int32(bconst)
                cand_s = cand_u ^ _I32_MIN
                acc = jnp.zeros((16,), jnp.int32)
                for c in range(16):
                    acc = acc + (kvs[c] >= cand_s).astype(jnp.int32)
                t_u = jnp.where(jnp.sum(acc) >= r1, cand_u, t_u)
            t_s = t_u ^ _I32_MIN
            a_gt = jnp.zeros((16,), jnp.int32)
            a_eq = jnp.zeros((16,), jnp.int32)
            for c in range(16):
                a_gt = a_gt + (kvs[c] > t_s).astype(jnp.int32)
                a_eq = a_eq + (kvs[c] == t_s).astype(jnp.int32)
            return t_s, jnp.sum(a_gt), jnp.sum(a_eq)

        def search_slow():
            for c in range(4):
                cbuf[pl.ds(n1 + 16 * c, 16)] = pad16
            nv4 = lax.shift_right_logical(n1 + 63, 6)

            def bs(i, t_u):
                cand_u = t_u | lax.shift_left(jnp.int32(1), 31 - i)
                cand_s = cand_u ^ _I32_MIN

                def cnt_body(j, acc):
                    for c in range(4):
                        kv = cbuf[pl.ds(j * 64 + c * 16, 16)]
                        acc = acc + (kv >= cand_s).astype(jnp.int32)
                    return acc

                acc = lax.fori_loop(0, nv4, cnt_body,
                                    jnp.zeros((16,), jnp.int32))
                return jnp.where(jnp.sum(acc) >= r1, cand_u, t_u)

            t_u = lax.fori_loop(0, 32, bs, jnp.int32(0))
            t_s = t_u ^ _I32_MIN

            def stats_body(j, accs):
                a_gt, a_eq = accs
                for c in range(4):
                    kv = cbuf[pl.ds(j * 64 + c * 16, 16)]
                    a_gt = a_gt + (kv > t_s).astype(jnp.int32)
                    a_eq = a_eq + (kv == t_s).astype(jnp.int32)
                return a_gt, a_eq

            z16 = jnp.zeros((16,), jnp.int32)
            a_gt, a_eq = lax.fori_loop(0, nv4, stats_body, (z16, z16))
            return t_s, jnp.sum(a_gt), jnp.sum(a_eq)

        t_s, c_gt, n_eq = lax.cond(n1 <= _FAST_CAP, search_fast, search_slow)
        e = r1 - c_gt

        @pl.when(e == n_eq)
        def _fast():
            @plsc.parallel_loop(0, _NV, unroll=_UNROLL)
            def out_b(j):
                sl = pl.ds(j * 16, 16)
                v = xv[sl]
                ov[sl] = jnp.where(_key_of(v) >= t_s, v, -jnp.inf)

        @pl.when(e != n_eq)
        def _slow():
            def out_b(j, run):
                sl = pl.ds(j * 16, 16)
                v = xv[sl]
                key = _key_of(v)
                eqm = key == t_s
                cs = plsc.cumsum(eqm.astype(jnp.int32))
                keep = (key > t_s) | (eqm & ((run + cs) <= e))
                ov[sl] = jnp.where(keep, v, -jnp.inf)
                return run + plsc.all_reduce_population_count(eqm)

            plsc.parallel_loop(0, _NV, unroll=_UNROLL,
                               carry=jnp.zeros((16,), jnp.int32))(out_b)

    pltpu.async_copy(row_slice(0), xv0, si0)

    def chunk(i, _):
        r0 = 2 * i
        pltpu.async_copy(row_slice(r0 + 1), xv1, si1)
        pltpu.make_async_copy(row_slice(r0), xv0, si0).wait()

        @pl.when(i > 0)
        def _w0():
            pltpu.make_async_copy(ov0, out_slice(2 * i - 2), so0).wait()

        select_row(xv0, ov0)
        pltpu.async_copy(ov0, out_slice(r0), so0)

        @pl.when(i < _RPW // 2 - 1)
        def _n1():
            pltpu.async_copy(row_slice(r0 + 2), xv0, si0)

        pltpu.make_async_copy(row_slice(r0 + 1), xv1, si1).wait()

        @pl.when(i > 0)
        def _w1():
            pltpu.make_async_copy(ov1, out_slice(2 * i - 1), so1).wait()

        select_row(xv1, ov1)
        pltpu.async_copy(ov1, out_slice(r0 + 1), so1)
        return 0

    lax.fori_loop(0, _RPW // 2, chunk, 0)
    pltpu.make_async_copy(ov0, out_slice(_RPW - 2), so0).wait()
    pltpu.make_async_copy(ov1, out_slice(_RPW - 1), so1).wait()


def _sc_topk_mask(flat):
    mesh = plsc.VectorSubcoreMesh(core_axis_name="c", subcore_axis_name="s")
    return pl.kernel(
        _sc_body,
        out_type=jax.ShapeDtypeStruct((_NROWS_SC * _V,), jnp.float32),
        mesh=mesh,
        compiler_params=pltpu.CompilerParams(needs_layout_passes=False),
        scratch_types=[
            pltpu.VMEM((_V,), jnp.float32),
            pltpu.VMEM((_V,), jnp.float32),
            pltpu.VMEM((_V,), jnp.float32),
            pltpu.VMEM((_V,), jnp.float32),
            pltpu.VMEM((_V + 64,), jnp.int32),
            pltpu.VMEM((_NBINS * 16,), jnp.int32),
            pltpu.SemaphoreType.DMA,
            pltpu.SemaphoreType.DMA,
            pltpu.SemaphoreType.DMA,
            pltpu.SemaphoreType.DMA,
        ],
    )(flat)


def _tc_body(x_ref, o_ref, *, k):
    x = x_ref[...]
    b = jax.lax.bitcast_convert_type(x, jnp.int32)
    key = jnp.where(b < 0, _I32_MIN - b, b)
    rows = x.shape[0]
    t_u = jnp.zeros((rows, 1), jnp.int32)

    def bit_step(i, t_u):
        bit = 31 - i
        cand_u = t_u | lax.shift_left(jnp.ones((), jnp.int32), bit)
        cand_s = cand_u ^ _I32_MIN
        cnt = jnp.sum((key >= cand_s).astype(jnp.int32), axis=1,
                      keepdims=True)
        return jnp.where(cnt >= k, cand_u, t_u)

    t_u = lax.fori_loop(0, 32, bit_step, t_u)
    t_s = t_u ^ _I32_MIN

    gt = key > t_s
    eq = key == t_s
    c_gt = jnp.sum(gt.astype(jnp.int32), axis=1, keepdims=True)
    e = k - c_gt

    idx = lax.broadcasted_iota(jnp.int32, x.shape, 1)
    eq_i = eq.astype(jnp.int32)
    t_i = jnp.zeros((rows, 1), jnp.int32)

    def idx_step(i, t_i):
        bit = 12 - i
        cand = t_i + lax.shift_left(jnp.ones((), jnp.int32), bit)
        cnt = jnp.sum(jnp.where(idx < cand, eq_i, 0), axis=1, keepdims=True)
        return jnp.where(cnt < e, cand, t_i)

    t_i = lax.fori_loop(0, 13, idx_step, t_i)
    keep = gt | (eq & (idx <= t_i))
    o_ref[...] = jnp.where(keep, x, -jnp.inf)


def _tc_topk_mask(x2d):
    n, V = x2d.shape
    rpb = 64
    return pl.pallas_call(
        lambda x_ref, o_ref: _tc_body(x_ref, o_ref, k=_K),
        grid=(n // rpb,),
        in_specs=[pl.BlockSpec((rpb, V), lambda i: (i, 0))],
        out_specs=pl.BlockSpec((rpb, V), lambda i: (i, 0)),
        out_shape=jax.ShapeDtypeStruct((n, V), jnp.float32),
    )(x2d)


@jax.jit
def _hybrid(x2d):
    sc_out = _sc_topk_mask(x2d[:_NROWS_SC].reshape(-1))
    tc_out = _tc_topk_mask(x2d[_NROWS_SC:])
    return jnp.concatenate([sc_out.reshape(_NROWS_SC, _V), tc_out], axis=0)


def kernel(logits):
    B, S, V = logits.shape
    out = _hybrid(logits.reshape(B * S, V))
    return out.reshape(B, S, V)


# speedup vs baseline: 2.8538x; 1.0007x over previous
import functools
import math

import jax
import jax.numpy as jnp
from jax import lax
from jax.experimental import pallas as pl
from jax.experimental.pallas import tpu as pltpu
from jax.experimental.pallas import tpu_sc as plsc

_I32_MIN = -(2**31)
_NROWS_SC = 256
_RPW = _NROWS_SC // 32
_V = 8192
_K = math.ceil((1.0 - 0.9) * _V)
_NV = _V // 16
_NBINS = 64
_NGRP = _NBINS // 16
_UNROLL = 8
_FAST_CAP = 240


def _digit(v):
    t = lax.convert_element_type(v * 8.0, jnp.int32)
    return jnp.clip(t + 32, 0, _NBINS - 1)


def _key_of(v):
    b = lax.bitcast_convert_type(v, jnp.int32)
    return jnp.where(b < 0, _I32_MIN - b, b)


def _sc_body(x_hbm, o_hbm, xv0, xv1, ov0, ov1, cbuf, hist,
             si0, si1, so0, so1):
    wid = lax.axis_index("s") * 2 + lax.axis_index("c")
    ii = lax.broadcasted_iota(jnp.int32, (16,), 0)
    ones16 = jnp.full((16,), 1, jnp.int32)
    pad16 = jnp.full((16,), _I32_MIN, jnp.int32)

    def row_slice(rr):
        return x_hbm.at[pl.ds((wid * _RPW + rr) * _V, _V)]

    def out_slice(rr):
        return o_hbm.at[pl.ds((wid * _RPW + rr) * _V, _V)]

    def select_row(xv, ov):
        for c in range(16):
            cbuf[pl.ds(c * 16, 16)] = pad16

        @plsc.parallel_loop(0, _NBINS * 16 // 16, unroll=_UNROLL)
        def zero_hist(i):
            hist[pl.ds(i * 16, 16)] = jnp.zeros((16,), jnp.int32)

        @plsc.parallel_loop(0, _NV, unroll=_UNROLL)
        def pass_a(j):
            v = xv[pl.ds(j * 16, 16)]
            addr = lax.shift_left(_digit(v), 4) + ii
            plsc.addupdate_scatter(hist, [addr], ones16)

        cum = jnp.int32(0)
        found = jnp.int32(0)
        b0 = jnp.int32(0)
        r1 = jnp.int32(1)
        for g in range(_NGRP - 1, -1, -1):
            mg = jnp.zeros((16,), jnp.int32)
            for b in range(16):
                s_b = jnp.sum(hist[pl.ds((16 * g + b) * 16, 16)])
                mg = jnp.where(ii == b, s_b, mg)
            rev = lax.rev(mg, (0,))
            cs = plsc.cumsum(rev)
            tot = jnp.max(cs)
            hit = cs >= (_K - cum)
            p = jnp.max(plsc.all_reduce_ffs(hit))
            in_this = jnp.logical_and(found == 0, cum + tot >= _K)
            cnt_d = jnp.sum(jnp.where(ii == p, rev, 0))
            cs_p = jnp.sum(jnp.where(ii == p, cs, 0))
            cum_above = cum + cs_p - cnt_d
            b0 = jnp.where(in_this, 16 * g + 15 - p, b0)
            r1 = jnp.where(in_this, _K - cum_above, r1)
            found = jnp.where(in_this, jnp.int32(1), found)
            cum = cum + tot

        def pass_c(j, ptr):
            v = xv[pl.ds(j * 16, 16)]
            m = _digit(v) == b0
            plsc.store_compressed(cbuf.at[pl.ds(ptr, 16)], _key_of(v),
                                  mask=m)
            return ptr + jnp.max(plsc.all_reduce_population_count(m))

        n1 = plsc.parallel_loop(0, _NV, unroll=_UNROLL,
                                carry=jnp.int32(0))(pass_c)

        def search_fast():
            kvs = [cbuf[pl.ds(c * 16, 16)] for c in range(16)]
            t_u = jnp.int32(0)
            for bit in range(31, -1, -1):
                bconst = -(1 << 31) if bit == 31 else (1 << bit)
                cand_u = t_u | jnp.int32(bconst)
                cand_s = cand_u ^ _I32_MIN
                acc = jnp.zeros((16,), jnp.int32)
                for c in range(16):
                    acc = acc + (kvs[c] >= cand_s).astype(jnp.int32)
                t_u = jnp.where(jnp.sum(acc) >= r1, cand_u, t_u)
            t_s = t_u ^ _I32_MIN
            a_gt = jnp.zeros((16,), jnp.int32)
            a_eq = jnp.zeros((16,), jnp.int32)
            for c in range(16):
                a_gt = a_gt + (kvs[c] > t_s).astype(jnp.int32)
                a_eq = a_eq + (kvs[c] == t_s).astype(jnp.int32)
            return t_s, jnp.sum(a_gt), jnp.sum(a_eq)

        def search_slow():
            for c in range(4):
                cbuf[pl.ds(n1 + 16 * c, 16)] = pad16
            nv4 = lax.shift_right_logical(n1 + 63, 6)

            def bs(i, t_u):
                cand_u = t_u | lax.shift_left(jnp.int32(1), 31 - i)
                cand_s = cand_u ^ _I32_MIN

                def cnt_body(j, acc):
                    for c in range(4):
                        kv = cbuf[pl.ds(j * 64 + c * 16, 16)]
                        acc = acc + (kv >= cand_s).astype(jnp.int32)
                    return acc

                acc = lax.fori_loop(0, nv4, cnt_body,
                                    jnp.zeros((16,), jnp.int32))
                return jnp.where(jnp.sum(acc) >= r1, cand_u, t_u)

            t_u = lax.fori_loop(0, 32, bs, jnp.int32(0))
            t_s = t_u ^ _I32_MIN

            def stats_body(j, accs):
                a_gt, a_eq = accs
                for c in range(4):
                    kv = cbuf[pl.ds(j * 64 + c * 16, 16)]
                    a_gt = a_gt + (kv > t_s).astype(jnp.int32)
                    a_eq = a_eq + (kv == t_s).astype(jnp.int32)
                return a_gt, a_eq

            z16 = jnp.zeros((16,), jnp.int32)
            a_gt, a_eq = lax.fori_loop(0, nv4, stats_body, (z16, z16))
            return t_s, jnp.sum(a_gt), jnp.sum(a_eq)

        t_s, c_gt, n_eq = lax.cond(n1 <= _FAST_CAP, search_fast, search_slow)
        e = r1 - c_gt

        @pl.when(e == n_eq)
        def _fast():
            @plsc.parallel_loop(0, _NV, unroll=_UNROLL)
            def out_b(j):
                sl = pl.ds(j * 16, 16)
                v = xv[sl]
                ov[sl] = jnp.where(_key_of(v) >= t_s, v, -jnp.inf)

        @pl.when(e != n_eq)
        def _slow():
            def out_b(j, run):
                sl = pl.ds(j * 16, 16)
                v = xv[sl]
                key = _key_of(v)
                eqm = key == t_s
                cs = plsc.cumsum(eqm.astype(jnp.int32))
                keep = (key > t_s) | (eqm & ((run + cs) <= e))
                ov[sl] = jnp.where(keep, v, -jnp.inf)
                return run + plsc.all_reduce_population_count(eqm)

            plsc.parallel_loop(0, _NV, unroll=_UNROLL,
                               carry=jnp.zeros((16,), jnp.int32))(out_b)

    pltpu.async_copy(row_slice(0), xv0, si0)

    def chunk(i, _):
        r0 = 2 * i
        pltpu.async_copy(row_slice(r0 + 1), xv1, si1)
        pltpu.make_async_copy(row_slice(r0), xv0, si0).wait()

        @pl.when(i > 0)
        def _w0():
            pltpu.make_async_copy(ov0, out_slice(2 * i - 2), so0).wait()

        select_row(xv0, ov0)
        pltpu.async_copy(ov0, out_slice(r0), so0)

        @pl.when(i < _RPW // 2 - 1)
        def _n1():
            pltpu.async_copy(row_slice(r0 + 2), xv0, si0)

        pltpu.make_async_copy(row_slice(r0 + 1), xv1, si1).wait()

        @pl.when(i > 0)
        def _w1():
            pltpu.make_async_copy(ov1, out_slice(2 * i - 1), so1).wait()

        select_row(xv1, ov1)
        pltpu.async_copy(ov1, out_slice(r0 + 1), so1)
        return 0

    lax.fori_loop(0, _RPW // 2, chunk, 0)
    pltpu.make_async_copy(ov0, out_slice(_RPW - 2), so0).wait()
    pltpu.make_async_copy(ov1, out_slice(_RPW - 1), so1).wait()


def _sc_topk_mask(flat):
    mesh = plsc.VectorSubcoreMesh(core_axis_name="c", subcore_axis_name="s")
    return pl.kernel(
        _sc_body,
        out_type=jax.ShapeDtypeStruct((_NROWS_SC * _V,), jnp.float32),
        mesh=mesh,
        compiler_params=pltpu.CompilerParams(needs_layout_passes=False),
        scratch_types=[
            pltpu.VMEM((_V,), jnp.float32),
            pltpu.VMEM((_V,), jnp.float32),
            pltpu.VMEM((_V,), jnp.float32),
            pltpu.VMEM((_V,), jnp.float32),
            pltpu.VMEM((_V + 64,), jnp.int32),
            pltpu.VMEM((_NBINS * 16,), jnp.int32),
            pltpu.SemaphoreType.DMA,
            pltpu.SemaphoreType.DMA,
            pltpu.SemaphoreType.DMA,
            pltpu.SemaphoreType.DMA,
        ],
    )(flat)


def _tc_body(x_ref, o_ref, *, k):
    x = x_ref[...]
    b = jax.lax.bitcast_convert_type(x, jnp.int32)
    key = jnp.where(b < 0, _I32_MIN - b, b)
    rows = x.shape[0]
    t_u = jnp.zeros((rows, 1), jnp.int32)

    def bit_step(i, t_u):
        bit = 31 - i
        cand_u = t_u | lax.shift_left(jnp.ones((), jnp.int32), bit)
        cand_s = cand_u ^ _I32_MIN
        cnt = jnp.sum((key >= cand_s).astype(jnp.int32), axis=1,
                      keepdims=True)
        return jnp.where(cnt >= k, cand_u, t_u)

    t_u = lax.fori_loop(0, 32, bit_step, t_u)
    t_s = t_u ^ _I32_MIN

    gt = key > t_s
    eq = key == t_s
    c_gt = jnp.sum(gt.astype(jnp.int32), axis=1, keepdims=True)
    e = k - c_gt

    idx = lax.broadcasted_iota(jnp.int32, x.shape, 1)
    eq_i = eq.astype(jnp.int32)
    t_i = jnp.zeros((rows, 1), jnp.int32)

    def idx_step(i, t_i):
        bit = 12 - i
        cand = t_i + lax.shift_left(jnp.ones((), jnp.int32), bit)
        cnt = jnp.sum(jnp.where(idx < cand, eq_i, 0), axis=1, keepdims=True)
        return jnp.where(cnt < e, cand, t_i)

    t_i = lax.fori_loop(0, 13, idx_step, t_i)
    keep = gt | (eq & (idx <= t_i))
    o_ref[...] = jnp.where(keep, x, -jnp.inf)


def _tc_topk_mask(x2d):
    n, V = x2d.shape
    rpb = 64
    return pl.pallas_call(
        lambda x_ref, o_ref: _tc_body(x_ref, o_ref, k=_K),
        grid=(n // rpb,),
        in_specs=[pl.BlockSpec((rpb, V), lambda i: (i, 0))],
        out_specs=pl.BlockSpec((rpb, V), lambda i: (i, 0)),
        out_shape=jax.ShapeDtypeStruct((n, V), jnp.float32),
    )(x2d)


@jax.jit
def _hybrid(x2d):
    sc_in = x2d[:_NROWS_SC].reshape(-1)
    tc_out = _tc_topk_mask(x2d[_NROWS_SC:])
    sc_out = _sc_topk_mask(sc_in)
    return jnp.concatenate([sc_out.reshape(_NROWS_SC, _V), tc_out], axis=0)


def kernel(logits):
    B, S, V = logits.shape
    out = _hybrid(logits.reshape(B * S, V))
    return out.reshape(B, S, V)
